# baseline (speedup 1.0000x reference)
import functools

import jax
import jax.numpy as jnp
import numpy as np
from jax import lax
from jax.experimental import pallas as pl
from jax.experimental.pallas import tpu as pltpu
from jax.experimental.pallas import tpu_sc as plsc

EPS = 1e-09
LANES = 16
RSQRT_MAGIC = np.int32(0x5F3759DF)


def _vrsqrt(v):
    y = plsc.bitcast(
        RSQRT_MAGIC - lax.shift_right_arithmetic(
            plsc.bitcast(v, jnp.int32), np.int32(1)),
        jnp.float32)
    for _ in range(3):
        y = y * (1.5 - 0.5 * v * y * y)
    return y


def _make_sc_fused(num_tokens, dim, chunk):
    info = plsc.get_sparse_core_info()
    nc, ns = info.num_cores, info.num_subcores
    nw = nc * ns
    per_w = num_tokens // nw
    n_chunks = per_w // chunk
    nslice = dim // LANES
    inv_dim = 1.0 / dim
    mesh = plsc.VectorSubcoreMesh(core_axis_name="c", subcore_axis_name="s")

    @functools.partial(
        pl.kernel,
        out_type=jax.ShapeDtypeStruct((num_tokens, dim), jnp.float32),
        mesh=mesh,
        compiler_params=pltpu.CompilerParams(needs_layout_passes=False),
        scratch_types=[
            pltpu.VMEM((per_w,), jnp.int32),
            pltpu.VMEM((chunk, dim), jnp.float32),
            pltpu.VMEM((chunk, dim), jnp.float32),
            pltpu.VMEM((chunk, dim), jnp.float32),
            pltpu.VMEM((chunk, dim), jnp.float32),
            pltpu.VMEM((dim,), jnp.float32),
            pltpu.VMEM((dim,), jnp.float32),
            pltpu.SemaphoreType.DMA,
            pltpu.SemaphoreType.DMA,
            pltpu.SemaphoreType.DMA,
            pltpu.SemaphoreType.DMA,
            pltpu.SemaphoreType.DMA,
            pltpu.SemaphoreType.DMA,
        ],
    )
    def sc_fused(ids_hbm, table_hbm, pos_hbm, g_hbm, b_hbm, out_hbm,
                 idx_v, w0, w1, p0, p1, g_v, b_v,
                 sg0, sg1, sp0, sp1, so0, so1):
        wid = lax.axis_index("s") * nc + lax.axis_index("c")
        base = wid * per_w
        seq = pos_hbm.shape[0]
        pos_base = base % seq
        pltpu.sync_copy(g_hbm, g_v)
        pltpu.sync_copy(b_hbm, b_v)
        pltpu.sync_copy(ids_hbm.at[pl.ds(base, per_w)], idx_v)
        wbufs, pbufs = (w0, w1), (p0, p1)
        gsems, psems, osems = (sg0, sg1), (sp0, sp1), (so0, so1)
        gathers = [None] * n_chunks
        ploads = [None] * n_chunks
        ostores = [None] * n_chunks

        def compute(k):
            wb = wbufs[k % 2]
            pb = pbufs[k % 2]
            gathers[k].wait()
            ploads[k].wait()

            @plsc.parallel_loop(0, chunk, 1, unroll=2)
            def token_body(t):
                accs = [jnp.zeros((LANES,), jnp.float32) for _ in range(4)]
                acc2s = [jnp.zeros((LANES,), jnp.float32) for _ in range(4)]
                for s in range(nslice):
                    sl = pl.ds(s * LANES, LANES)
                    x = wb[t, sl] + pb[t, sl]
                    wb[t, sl] = x
                    accs[s % 4] = accs[s % 4] + x
                    acc2s[s % 4] = acc2s[s % 4] + x * x
                acc = (accs[0] + accs[1]) + (accs[2] + accs[3])
                acc2 = (acc2s[0] + acc2s[1]) + (acc2s[2] + acc2s[3])
                mean = jnp.sum(acc) * inv_dim
                msq = jnp.sum(acc2) * inv_dim
                var = msq - mean * mean + EPS
                rstd = _vrsqrt(jnp.full((LANES,), var, jnp.float32))
                shift = jnp.full((LANES,), -mean, jnp.float32) * rstd
                for s in range(nslice):
                    sl = pl.ds(s * LANES, LANES)
                    wb[t, sl] = (wb[t, sl] * rstd + shift) * g_v[sl] + b_v[sl]

            ostores[k] = pltpu.async_copy(
                wb, out_hbm.at[pl.ds(base + k * chunk, chunk)], osems[k % 2])

        for c in range(n_chunks):
            if c >= 2:
                ostores[c - 2].wait()
            gathers[c] = pltpu.async_copy(
                table_hbm.at[idx_v.at[pl.ds(c * chunk, chunk)]],
                wbufs[c % 2], gsems[c % 2])
            ploads[c] = pltpu.async_copy(
                pos_hbm.at[pl.ds(pos_base + c * chunk, chunk)],
                pbufs[c % 2], psems[c % 2])
            if c >= 1:
                compute(c - 1)
        compute(n_chunks - 1)
        ostores[n_chunks - 2].wait()
        ostores[n_chunks - 1].wait()

    return sc_fused


def _tc_pos_body(p_ref, out_ref):
    p = p_ref[...]
    out_ref[...] = jnp.broadcast_to(p[None], out_ref.shape)


def _tc_pos(pos_table, b, sblk):
    n, d = pos_table.shape
    return pl.pallas_call(
        _tc_pos_body,
        grid=(n // sblk,),
        in_specs=[pl.BlockSpec((sblk, d), lambda j: (j, 0))],
        out_specs=pl.BlockSpec((b, sblk, d), lambda j: (0, j, 0)),
        out_shape=jax.ShapeDtypeStruct((b, n, d), jnp.float32),
    )(pos_table)


def kernel(input_ids, word_table, pos_table, ln_gamma, ln_beta):
    b, n = input_ids.shape
    d = word_table.shape[1]
    ids = input_ids.reshape(-1).astype(jnp.int32)
    out = _make_sc_fused(b * n, d, 32)(
        ids, word_table, pos_table, ln_gamma, ln_beta)
    pos_out = _tc_pos(pos_table, b, 2048)
    return out.reshape(b, n, d), pos_out


# speedup vs baseline: 1.5435x; 1.5435x over previous
import functools

import jax
import jax.numpy as jnp
from jax import lax
from jax.experimental import pallas as pl
from jax.experimental.pallas import tpu as pltpu
from jax.experimental.pallas import tpu_sc as plsc

EPS = 1e-09


def _make_sc_gather(num_tokens, dim, chunk):
    info = plsc.get_sparse_core_info()
    nc, ns = info.num_cores, info.num_subcores
    nw = nc * ns
    per_w = num_tokens // nw
    n_chunks = per_w // chunk
    mesh = plsc.VectorSubcoreMesh(core_axis_name="c", subcore_axis_name="s")

    @functools.partial(
        pl.kernel,
        out_type=jax.ShapeDtypeStruct((num_tokens, dim), jnp.float32),
        mesh=mesh,
        scratch_types=[
            pltpu.VMEM((per_w,), jnp.int32),
            pltpu.VMEM((chunk, dim), jnp.float32),
            pltpu.VMEM((chunk, dim), jnp.float32),
            pltpu.SemaphoreType.DMA,
            pltpu.SemaphoreType.DMA,
        ],
    )
    def sc_gather(ids_hbm, table_hbm, out_hbm, idx_v, buf0, buf1, sem0, sem1):
        wid = lax.axis_index("s") * nc + lax.axis_index("c")
        base = wid * per_w
        pltpu.sync_copy(ids_hbm.at[pl.ds(base, per_w)], idx_v)
        bufs = (buf0, buf1)
        sems = (sem0, sem1)
        copies = [None] * n_chunks
        for c in range(n_chunks):
            copies[c] = pltpu.async_copy(
                table_hbm.at[idx_v.at[pl.ds(c * chunk, chunk)]],
                bufs[c % 2],
                sems[c % 2],
            )
            if c >= 1:
                copies[c - 1].wait()
                pltpu.sync_copy(
                    bufs[(c - 1) % 2],
                    out_hbm.at[pl.ds(base + (c - 1) * chunk, chunk)],
                )
        copies[n_chunks - 1].wait()
        pltpu.sync_copy(
            bufs[(n_chunks - 1) % 2],
            out_hbm.at[pl.ds(base + (n_chunks - 1) * chunk, chunk)],
        )

    return sc_gather


def _make_sc_pos(batch, seq, dim, chunk):
    info = plsc.get_sparse_core_info()
    nc, ns = info.num_cores, info.num_subcores
    nw = nc * ns
    num_tokens = batch * seq
    per_w = num_tokens // nw
    n_chunks = per_w // chunk
    mesh = plsc.VectorSubcoreMesh(core_axis_name="c", subcore_axis_name="s")

    @functools.partial(
        pl.kernel,
        out_type=jax.ShapeDtypeStruct((num_tokens, dim), jnp.float32),
        mesh=mesh,
        scratch_types=[
            pltpu.VMEM((chunk, dim), jnp.float32),
            pltpu.VMEM((chunk, dim), jnp.float32),
            pltpu.SemaphoreType.DMA,
            pltpu.SemaphoreType.DMA,
            pltpu.SemaphoreType.DMA,
            pltpu.SemaphoreType.DMA,
        ],
    )
    def sc_pos(gathered_hbm, pos_hbm, out_hbm, buf0, buf1,
               li0, li1, so0, so1):
        del gathered_hbm
        wid = lax.axis_index("s") * nc + lax.axis_index("c")
        base = wid * per_w
        pos_base = base % seq
        bufs = (buf0, buf1)
        lsems = (li0, li1)
        osems = (so0, so1)
        loads = [None] * n_chunks
        stores = [None] * n_chunks
        for c in range(n_chunks):
            if c >= 2:
                stores[c - 2].wait()
            loads[c] = pltpu.async_copy(
                pos_hbm.at[pl.ds(pos_base + c * chunk, chunk)],
                bufs[c % 2], lsems[c % 2])
            if c >= 1:
                loads[c - 1].wait()
                stores[c - 1] = pltpu.async_copy(
                    bufs[(c - 1) % 2],
                    out_hbm.at[pl.ds(base + (c - 1) * chunk, chunk)],
                    osems[(c - 1) % 2])
        loads[n_chunks - 1].wait()
        stores[n_chunks - 1] = pltpu.async_copy(
            bufs[(n_chunks - 1) % 2],
            out_hbm.at[pl.ds(base + (n_chunks - 1) * chunk, chunk)],
            osems[(n_chunks - 1) % 2])
        stores[n_chunks - 2].wait()
        stores[n_chunks - 1].wait()

    return sc_pos


def _tc_ln_body(w_ref, p_ref, g_ref, b_ref, out_ref):
    w = w_ref[0]
    p = p_ref[...]
    x = w + p
    d = x.shape[-1]
    s1 = jnp.sum(x, axis=-1, keepdims=True)
    s2 = jnp.sum(x * x, axis=-1, keepdims=True)
    mean = s1 * (1.0 / d)
    var = s2 * (1.0 / d) - mean * mean
    rstd = lax.rsqrt(var + EPS)
    shift = -mean * rstd
    out_ref[0] = (x * rstd + shift) * g_ref[...] + b_ref[...]


def _tc_ln(w3, pos_table, gamma, beta, sblk):
    b, n, d = w3.shape
    grid = (n // sblk, b)
    return pl.pallas_call(
        _tc_ln_body,
        grid=grid,
        in_specs=[
            pl.BlockSpec((1, sblk, d), lambda j, i: (i, j, 0)),
            pl.BlockSpec((sblk, d), lambda j, i: (j, 0)),
            pl.BlockSpec((1, d), lambda j, i: (0, 0)),
            pl.BlockSpec((1, d), lambda j, i: (0, 0)),
        ],
        out_specs=pl.BlockSpec((1, sblk, d), lambda j, i: (i, j, 0)),
        out_shape=jax.ShapeDtypeStruct((b, n, d), jnp.float32),
    )(w3, pos_table, gamma.reshape(1, d), beta.reshape(1, d))


def kernel(input_ids, word_table, pos_table, ln_gamma, ln_beta):
    b, n = input_ids.shape
    d = word_table.shape[1]
    ids = input_ids.reshape(-1).astype(jnp.int32)
    gathered = _make_sc_gather(b * n, d, 64)(ids, word_table)
    pos_out = _make_sc_pos(b, n, d, 64)(gathered, pos_table)
    w3 = gathered.reshape(b, n, d)
    out = _tc_ln(w3, pos_table, ln_gamma, ln_beta, 2048)
    return out, pos_out.reshape(b, n, d)


# speedup vs baseline: 1.7060x; 1.1053x over previous
import functools

import jax
import jax.numpy as jnp
from jax import lax
from jax.experimental import pallas as pl
from jax.experimental.pallas import tpu as pltpu
from jax.experimental.pallas import tpu_sc as plsc

EPS = 1e-09


def _make_sc_gather(num_tokens, dim, chunk):
    info = plsc.get_sparse_core_info()
    nc, ns = info.num_cores, info.num_subcores
    nw = nc * ns
    per_w = num_tokens // nw
    n_chunks = per_w // chunk
    mesh = plsc.VectorSubcoreMesh(core_axis_name="c", subcore_axis_name="s")

    @functools.partial(
        pl.kernel,
        out_type=jax.ShapeDtypeStruct((num_tokens, dim), jnp.float32),
        mesh=mesh,
        scratch_types=[
            pltpu.VMEM((per_w,), jnp.int32),
            pltpu.VMEM((chunk, dim), jnp.float32),
            pltpu.VMEM((chunk, dim), jnp.float32),
            pltpu.SemaphoreType.DMA,
            pltpu.SemaphoreType.DMA,
        ],
    )
    def sc_gather(ids_hbm, table_hbm, out_hbm, idx_v, buf0, buf1, sem0, sem1):
        wid = lax.axis_index("s") * nc + lax.axis_index("c")
        base = wid * per_w
        pltpu.sync_copy(ids_hbm.at[pl.ds(base, per_w)], idx_v)
        bufs = (buf0, buf1)
        sems = (sem0, sem1)
        copies = [None] * n_chunks
        for c in range(n_chunks):
            copies[c] = pltpu.async_copy(
                table_hbm.at[idx_v.at[pl.ds(c * chunk, chunk)]],
                bufs[c % 2],
                sems[c % 2],
            )
            if c >= 1:
                copies[c - 1].wait()
                pltpu.sync_copy(
                    bufs[(c - 1) % 2],
                    out_hbm.at[pl.ds(base + (c - 1) * chunk, chunk)],
                )
        copies[n_chunks - 1].wait()
        pltpu.sync_copy(
            bufs[(n_chunks - 1) % 2],
            out_hbm.at[pl.ds(base + (n_chunks - 1) * chunk, chunk)],
        )

    return sc_gather


def _make_sc_pos(batch, seq, dim):
    info = plsc.get_sparse_core_info()
    nc, ns = info.num_cores, info.num_subcores
    nw = nc * ns
    num_tokens = batch * seq
    per_w = seq // nw
    mesh = plsc.VectorSubcoreMesh(core_axis_name="c", subcore_axis_name="s")

    @functools.partial(
        pl.kernel,
        out_type=jax.ShapeDtypeStruct((num_tokens, dim), jnp.float32),
        mesh=mesh,
        scratch_types=[
            pltpu.VMEM((per_w, dim), jnp.float32),
            pltpu.SemaphoreType.DMA,
        ],
    )
    def sc_pos(gathered_hbm, pos_hbm, out_hbm, buf, sem):
        del gathered_hbm
        wid = lax.axis_index("s") * nc + lax.axis_index("c")
        pos_base = wid * per_w
        pltpu.sync_copy(pos_hbm.at[pl.ds(pos_base, per_w)], buf)
        stores = [
            pltpu.async_copy(
                buf, out_hbm.at[pl.ds(bi * seq + pos_base, per_w)], sem)
            for bi in range(batch)
        ]
        for st in stores:
            st.wait()

    return sc_pos


def _tc_ln_body(w_ref, p_ref, g_ref, b_ref, out_ref):
    w = w_ref[0]
    p = p_ref[...]
    x = w + p
    d = x.shape[-1]
    s1 = jnp.sum(x, axis=-1, keepdims=True)
    s2 = jnp.sum(x * x, axis=-1, keepdims=True)
    mean = s1 * (1.0 / d)
    var = s2 * (1.0 / d) - mean * mean
    rstd = lax.rsqrt(var + EPS)
    shift = -mean * rstd
    out_ref[0] = (x * rstd + shift) * g_ref[...] + b_ref[...]


def _tc_ln(w3, pos_table, gamma, beta, sblk):
    b, n, d = w3.shape
    grid = (n // sblk, b)
    return pl.pallas_call(
        _tc_ln_body,
        grid=grid,
        in_specs=[
            pl.BlockSpec((1, sblk, d), lambda j, i: (i, j, 0)),
            pl.BlockSpec((sblk, d), lambda j, i: (j, 0)),
            pl.BlockSpec((1, d), lambda j, i: (0, 0)),
            pl.BlockSpec((1, d), lambda j, i: (0, 0)),
        ],
        out_specs=pl.BlockSpec((1, sblk, d), lambda j, i: (i, j, 0)),
        out_shape=jax.ShapeDtypeStruct((b, n, d), jnp.float32),
    )(w3, pos_table, gamma.reshape(1, d), beta.reshape(1, d))


def kernel(input_ids, word_table, pos_table, ln_gamma, ln_beta):
    b, n = input_ids.shape
    d = word_table.shape[1]
    ids = input_ids.reshape(-1).astype(jnp.int32)
    gathered = _make_sc_gather(b * n, d, 64)(ids, word_table)
    pos_out = _make_sc_pos(b, n, d)(gathered, pos_table)
    w3 = gathered.reshape(b, n, d)
    out = _tc_ln(w3, pos_table, ln_gamma, ln_beta, 2048)
    return out, pos_out.reshape(b, n, d)


# speedup vs baseline: 1.7776x; 1.0420x over previous
import functools

import jax
import jax.numpy as jnp
from jax import lax
from jax.experimental import pallas as pl
from jax.experimental.pallas import tpu as pltpu
from jax.experimental.pallas import tpu_sc as plsc

EPS = 1e-09


def _make_sc_gather(num_tokens, dim, chunk):
    info = plsc.get_sparse_core_info()
    nc, ns = info.num_cores, info.num_subcores
    nw = nc * ns
    per_w = num_tokens // nw
    n_chunks = per_w // chunk
    mesh = plsc.VectorSubcoreMesh(core_axis_name="c", subcore_axis_name="s")

    @functools.partial(
        pl.kernel,
        out_type=jax.ShapeDtypeStruct((num_tokens, dim), jnp.float32),
        mesh=mesh,
        scratch_types=[
            pltpu.VMEM((per_w,), jnp.int32),
            pltpu.VMEM((chunk, dim), jnp.float32),
            pltpu.VMEM((chunk, dim), jnp.float32),
            pltpu.SemaphoreType.DMA,
            pltpu.SemaphoreType.DMA,
        ],
    )
    def sc_gather(ids_hbm, table_hbm, out_hbm, idx_v, buf0, buf1, sem0, sem1):
        wid = lax.axis_index("s") * nc + lax.axis_index("c")
        base = wid * per_w
        pltpu.sync_copy(ids_hbm.at[pl.ds(base, per_w)], idx_v)
        bufs = (buf0, buf1)
        sems = (sem0, sem1)
        copies = [None] * n_chunks
        for c in range(n_chunks):
            copies[c] = pltpu.async_copy(
                table_hbm.at[idx_v.at[pl.ds(c * chunk, chunk)]],
                bufs[c % 2],
                sems[c % 2],
            )
            if c >= 1:
                copies[c - 1].wait()
                pltpu.sync_copy(
                    bufs[(c - 1) % 2],
                    out_hbm.at[pl.ds(base + (c - 1) * chunk, chunk)],
                )
        copies[n_chunks - 1].wait()
        pltpu.sync_copy(
            bufs[(n_chunks - 1) % 2],
            out_hbm.at[pl.ds(base + (n_chunks - 1) * chunk, chunk)],
        )

    return sc_gather


def _tc_ln_body(w_ref, p_ref, g_ref, b_ref, out_ref):
    w = w_ref[0]
    p = p_ref[...]
    x = w + p
    mean = jnp.mean(x, axis=-1, keepdims=True)
    xc = x - mean
    var = jnp.mean(xc * xc, axis=-1, keepdims=True)
    xhat = xc * lax.rsqrt(var + EPS)
    out_ref[0] = xhat * g_ref[...] + b_ref[...]


def _tc_ln(w3, pos_table, gamma, beta, sblk):
    b, n, d = w3.shape
    grid = (n // sblk, b)
    return pl.pallas_call(
        _tc_ln_body,
        grid=grid,
        in_specs=[
            pl.BlockSpec((1, sblk, d), lambda j, i: (i, j, 0)),
            pl.BlockSpec((sblk, d), lambda j, i: (j, 0)),
            pl.BlockSpec((1, d), lambda j, i: (0, 0)),
            pl.BlockSpec((1, d), lambda j, i: (0, 0)),
        ],
        out_specs=pl.BlockSpec((1, sblk, d), lambda j, i: (i, j, 0)),
        out_shape=jax.ShapeDtypeStruct((b, n, d), jnp.float32),
    )(w3, pos_table, gamma.reshape(1, d), beta.reshape(1, d))


def _tc_pos_body(p_ref, out_ref):
    p = p_ref[...]
    out_ref[...] = jnp.broadcast_to(p[None], out_ref.shape)


def _tc_pos(pos_table, b, sblk):
    n, d = pos_table.shape
    return pl.pallas_call(
        _tc_pos_body,
        grid=(n // sblk,),
        in_specs=[pl.BlockSpec((sblk, d), lambda j: (j, 0))],
        out_specs=pl.BlockSpec((b, sblk, d), lambda j: (0, j, 0)),
        out_shape=jax.ShapeDtypeStruct((b, n, d), jnp.float32),
    )(pos_table)


def kernel(input_ids, word_table, pos_table, ln_gamma, ln_beta):
    b, n = input_ids.shape
    d = word_table.shape[1]
    ids = input_ids.reshape(-1).astype(jnp.int32)
    gathered = _make_sc_gather(b * n, d, 64)(ids, word_table)
    pos_out = _tc_pos(pos_table, b, 2048)
    w3 = gathered.reshape(b, n, d)
    out = _tc_ln(w3, pos_table, ln_gamma, ln_beta, 2048)
    return out, pos_out
